# Initial kernel scaffold; baseline (speedup 1.0000x reference)
import jax
import jax.numpy as jnp
from jax.experimental import pallas as pl


def kernel(x, edge_index, W1, b1, W2, b2):
    raise NotImplementedError("write your pallas kernel here")


# speedup vs baseline: 8.1760x; 8.1760x over previous
import functools

import jax
import jax.numpy as jnp
from jax import lax
from jax.experimental import pallas as pl
from jax.experimental.pallas import tpu as pltpu
from jax.experimental.pallas import tpu_sc as plsc

N = 10000
F = 128
E = 320000
NW = 32
LPT = 16
NPAD = 10240
NROWS = NPAD // 128
EPT_DEG = E // NW
CHUNKS = 80
EPAD = NW * CHUNKS * 128
BLK = 1024

_mesh = plsc.VectorSubcoreMesh(core_axis_name="c", subcore_axis_name="s")


@functools.partial(
    pl.kernel,
    mesh=_mesh,
    compiler_params=pltpu.CompilerParams(needs_layout_passes=False),
    out_type=jax.ShapeDtypeStruct((2 * NPAD,), jnp.float32),
    scratch_types=[
        pltpu.VMEM((EPT_DEG,), jnp.int32),
        pltpu.VMEM((NPAD,), jnp.float32),
        pltpu.VMEM((NPAD // 16,), jnp.float32),
        pltpu.VMEM((NPAD // 16,), jnp.float32),
        pltpu.VMEM_SHARED((16, NPAD), jnp.float32),
    ],
)
def _deg_kernel(row_hbm, out_hbm, idxbuf, hist, tmp, accb, shared):
    c = lax.axis_index("c")
    s = lax.axis_index("s")
    wid = c * 16 + s
    npt = NPAD // 16

    def _zero(i, _):
        hist[pl.ds(i * LPT, LPT)] = jnp.zeros((LPT,), jnp.float32)
        return 0
    lax.fori_loop(0, NPAD // LPT, _zero, 0)

    pltpu.sync_copy(row_hbm.at[pl.ds(wid * EPT_DEG, EPT_DEG)], idxbuf)
    ones = jnp.ones((LPT,), jnp.float32)

    def _acc(i, _):
        v = idxbuf[pl.ds(i * LPT, LPT)]
        plsc.addupdate_scatter(hist, [v], ones)
        return 0
    lax.fori_loop(0, EPT_DEG // LPT, _acc, 0)

    pltpu.sync_copy(hist, shared.at[s])
    plsc.subcore_barrier()

    def _zacc(i, _):
        accb[pl.ds(i * LPT, LPT)] = jnp.zeros((LPT,), jnp.float32)
        return 0
    lax.fori_loop(0, npt // LPT, _zacc, 0)

    def _red(t, _):
        pltpu.sync_copy(shared.at[t, pl.ds(s * npt, npt)], tmp)

        def _add(i, _):
            sl = pl.ds(i * LPT, LPT)
            accb[sl] = accb[sl] + tmp[sl]
            return 0
        lax.fori_loop(0, npt // LPT, _add, 0)
        return 0
    lax.fori_loop(0, 16, _red, 0)

    pltpu.sync_copy(accb, out_hbm.at[pl.ds(c * NPAD + s * npt, npt)])


@functools.partial(
    pl.kernel,
    mesh=_mesh,
    out_type=jax.ShapeDtypeStruct((2, NPAD, F), jnp.float32),
    scratch_types=[
        pltpu.VMEM((CHUNKS, 128), jnp.int32),
        pltpu.VMEM((CHUNKS, 128), jnp.int32),
        pltpu.VMEM((128, F), jnp.float32),
        pltpu.VMEM_SHARED((NPAD, F), jnp.float32),
        pltpu.SemaphoreType.DMA,
    ],
)
def _prop_kernel(y_hbm, row_hbm, col_hbm, zeros_hbm,
                 out_hbm, rowbuf, colbuf, gbuf, acc, gsem):
    c = lax.axis_index("c")
    s = lax.axis_index("s")
    wid = c * 16 + s
    rows_per_tile = NPAD // 16

    pltpu.sync_copy(zeros_hbm, acc.at[pl.ds(s * rows_per_tile, rows_per_tile)])

    pltpu.sync_copy(row_hbm.at[pl.ds(wid * CHUNKS, CHUNKS)], rowbuf)
    pltpu.sync_copy(col_hbm.at[pl.ds(wid * CHUNKS, CHUNKS)], colbuf)
    plsc.subcore_barrier()

    def _edge_chunk(j, _):
        pltpu.async_copy(y_hbm.at[rowbuf.at[j]], gbuf, gsem).wait()
        pltpu.sync_copy(gbuf, acc.at[colbuf.at[j]], add=True)
        return 0
    lax.fori_loop(0, CHUNKS, _edge_chunk, 0)
    plsc.subcore_barrier()

    pltpu.sync_copy(acc.at[pl.ds(s * rows_per_tile, rows_per_tile)],
                    out_hbm.at[c, pl.ds(s * rows_per_tile, rows_per_tile)])


def _lin1_body(degp_ref, x_ref, w_ref, b_ref, y_ref, dis_ref):
    deg = degp_ref[0] + degp_ref[1] + 1.0
    dis = lax.rsqrt(deg)
    y = lax.dot_general(x_ref[...], w_ref[...], (((1,), (1,)), ((), ())),
                        preferred_element_type=jnp.float32) + b_ref[...]
    y_ref[...] = y * dis
    dis_ref[...] = dis


def _lin2_body(p_ref, y1_ref, dis_ref, w_ref, b_ref, y2_ref):
    dis = dis_ref[...]
    x2 = jnp.maximum((p_ref[0] + p_ref[1] + y1_ref[...]) * dis, 0.0)
    y2 = lax.dot_general(x2, w_ref[...], (((1,), (1,)), ((), ())),
                         preferred_element_type=jnp.float32) + b_ref[...]
    y2_ref[...] = y2 * dis


def _final_body(p_ref, y2_ref, dis_ref, o_ref):
    o_ref[...] = jnp.maximum(
        (p_ref[0] + p_ref[1] + y2_ref[...]) * dis_ref[...], 0.0)


def _lin1(degp, xp, W1, b1):
    return pl.pallas_call(
        _lin1_body,
        grid=(NPAD // BLK,),
        in_specs=[
            pl.BlockSpec((2, BLK, 1), lambda i: (0, i, 0)),
            pl.BlockSpec((BLK, F), lambda i: (i, 0)),
            pl.BlockSpec((F, F), lambda i: (0, 0)),
            pl.BlockSpec((1, F), lambda i: (0, 0)),
        ],
        out_specs=[
            pl.BlockSpec((BLK, F), lambda i: (i, 0)),
            pl.BlockSpec((BLK, 1), lambda i: (i, 0)),
        ],
        out_shape=[
            jax.ShapeDtypeStruct((NPAD, F), jnp.float32),
            jax.ShapeDtypeStruct((NPAD, 1), jnp.float32),
        ],
    )(degp, xp, W1, b1.reshape(1, F))


def _lin2(p, y1, dis, W2, b2):
    return pl.pallas_call(
        _lin2_body,
        grid=(NPAD // BLK,),
        in_specs=[
            pl.BlockSpec((2, BLK, F), lambda i: (0, i, 0)),
            pl.BlockSpec((BLK, F), lambda i: (i, 0)),
            pl.BlockSpec((BLK, 1), lambda i: (i, 0)),
            pl.BlockSpec((F, F), lambda i: (0, 0)),
            pl.BlockSpec((1, F), lambda i: (0, 0)),
        ],
        out_specs=pl.BlockSpec((BLK, F), lambda i: (i, 0)),
        out_shape=jax.ShapeDtypeStruct((NPAD, F), jnp.float32),
    )(p, y1, dis, W2, b2.reshape(1, F))


def _final(p, y2, dis):
    return pl.pallas_call(
        _final_body,
        grid=(NPAD // BLK,),
        in_specs=[
            pl.BlockSpec((2, BLK, F), lambda i: (0, i, 0)),
            pl.BlockSpec((BLK, F), lambda i: (i, 0)),
            pl.BlockSpec((BLK, 1), lambda i: (i, 0)),
        ],
        out_specs=pl.BlockSpec((BLK, F), lambda i: (i, 0)),
        out_shape=jax.ShapeDtypeStruct((NPAD, F), jnp.float32),
    )(p, y2, dis)


def kernel(x, edge_index, W1, b1, W2, b2):
    row = edge_index[0].astype(jnp.int32)
    col = edge_index[1].astype(jnp.int32)

    npad_e = EPAD - E
    rowp = jnp.concatenate([row, jnp.zeros((npad_e,), jnp.int32)])
    colp = jnp.concatenate([col, jnp.full((npad_e,), NPAD - 1, jnp.int32)])
    rowp = rowp.reshape(NW * CHUNKS, 128)
    colp = colp.reshape(NW * CHUNKS, 128)

    degp = _deg_kernel(row).reshape(2, NPAD, 1)

    xp = jnp.pad(x[0], ((0, NPAD - N), (0, 0)))
    zeros = jnp.zeros((NPAD // 16, F), jnp.float32)

    y1, dis = _lin1(degp, xp, W1, b1)
    p1 = _prop_kernel(y1, rowp, colp, zeros)
    y2 = _lin2(p1, y1, dis, W2, b2)
    p2 = _prop_kernel(y2, rowp, colp, zeros)
    out = _final(p2, y2, dis)
    return out[:N].reshape(1, N, F)
